# baseline (device time: 36177 ns/iter reference)
import jax
import jax.numpy as jnp
from jax import lax
from jax.experimental import pallas as pl
from jax.experimental.pallas import tpu as pltpu

N_DEV = 4
B, SQ, D = 2, 256, 768
HQ_PER, DH = 8, 64
KV_PER = 2
R = B * SQ
DHALF = D // 2

_MESH = pl.DeviceIdType.MESH


def kernel(x, Wq, Wo, Wk, Wv):
    def body(x_ref, wq_ref, wo_ref, wk_ref, wv_ref, out_ref,
             attn_ref, acc_ref, land_a1, land_b1, land_a2, land_b2,
             send_sems, recv_sems):
        p = lax.axis_index("i")
        cx = lax.shift_right_logical(p, 1)
        cy = lax.bitwise_and(lax.bitwise_xor(p, cx), 1)
        py = lax.bitwise_xor(p, 1)
        px = lax.bitwise_xor(p, 3)

        barrier_sem = pltpu.get_barrier_semaphore()
        for nbr in (py, px):
            pl.semaphore_signal(barrier_sem, inc=1, device_id=(nbr,),
                                device_id_type=_MESH)
        pl.semaphore_wait(barrier_sem, 2)

        x2 = x_ref[:].reshape(R, D).astype(jnp.bfloat16)
        wqb = (wq_ref[:] * 0.125).astype(jnp.bfloat16)
        q = jnp.dot(x2, wqb, preferred_element_type=jnp.float32)
        wk_sl = wk_ref[:, pl.ds(p * (KV_PER * DH), KV_PER * DH)]
        wv_sl = wv_ref[:, pl.ds(p * (KV_PER * DH), KV_PER * DH)]
        k = jnp.dot(x2, wk_sl.astype(jnp.bfloat16),
                    preferred_element_type=jnp.float32)
        v = jnp.dot(x2, wv_sl.astype(jnp.bfloat16),
                    preferred_element_type=jnp.float32)

        for b in range(B):
            rows = slice(b * SQ, (b + 1) * SQ)
            for h in range(HQ_PER):
                g = h // 4
                qh = q[rows, h * DH:(h + 1) * DH].astype(jnp.bfloat16)
                kh = k[rows, g * DH:(g + 1) * DH].astype(jnp.bfloat16)
                vh = v[rows, g * DH:(g + 1) * DH].astype(jnp.bfloat16)
                s = lax.dot_general(
                    qh, kh, (((1,), (1,)), ((), ())),
                    preferred_element_type=jnp.float32)
                e = jnp.exp(s)
                denom = jnp.sum(e, axis=1, keepdims=True)
                o = jnp.dot(e.astype(jnp.bfloat16), vh,
                            preferred_element_type=jnp.float32) / denom
                attn_ref[rows, h * DH:(h + 1) * DH] = o.astype(jnp.bfloat16)

        wob = wo_ref[:].astype(jnp.bfloat16)

        def wo_quadrant(row_off, col_off):
            block = attn_ref[pl.ds(row_off, SQ), :]
            return jnp.dot(block, wob[:, col_off:col_off + DHALF],
                           preferred_element_type=jnp.float32)

        def exchange(src_off, n_rows, col_off, dst_ref, partner, sem):
            rdma = pltpu.make_async_remote_copy(
                src_ref=acc_ref.at[pl.ds(src_off, n_rows),
                                   pl.ds(col_off, DHALF)],
                dst_ref=dst_ref,
                send_sem=send_sems.at[sem],
                recv_sem=recv_sems.at[sem],
                device_id=(partner,),
                device_id_type=_MESH,
            )
            rdma.start()
            return rdma

        def acc_add(row_off, n_rows, col_off, land):
            acc_ref[pl.ds(row_off, n_rows), pl.ds(col_off, DHALF)] = (
                acc_ref[pl.ds(row_off, n_rows), pl.ds(col_off, DHALF)]
                + land[:, :]
            )

        a_keep1, a_send1 = cy * SQ, (1 - cy) * SQ
        b_keep1, b_send1 = cx * SQ, (1 - cx) * SQ
        a_keep2, a_send2 = a_keep1 + cx * 128, a_keep1 + (1 - cx) * 128
        b_keep2, b_send2 = b_keep1 + cy * 128, b_keep1 + (1 - cy) * 128

        acc_ref[pl.ds(a_send1, SQ), pl.ds(0, DHALF)] = wo_quadrant(a_send1, 0)
        ra = exchange(a_send1, SQ, 0, land_a1, py, 0)
        acc_ref[pl.ds(b_send1, SQ), pl.ds(DHALF, DHALF)] = (
            wo_quadrant(b_send1, DHALF))
        rb = exchange(b_send1, SQ, DHALF, land_b1, px, 4)
        keep_a = wo_quadrant(a_keep1, 0)
        keep_b = wo_quadrant(b_keep1, DHALF)
        ra.wait()
        rb.wait()
        acc_ref[pl.ds(a_keep1, SQ), pl.ds(0, DHALF)] = keep_a + land_a1[:, :]
        acc_ref[pl.ds(b_keep1, SQ), pl.ds(DHALF, DHALF)] = (
            keep_b + land_b1[:, :])

        ra = exchange(a_send2, 128, 0, land_a2, px, 1)
        rb = exchange(b_send2, 128, DHALF, land_b2, py, 5)
        ra.wait()
        rb.wait()
        acc_add(a_keep2, 128, 0, land_a2)
        acc_add(b_keep2, 128, DHALF, land_b2)

        ra = exchange(a_keep2, 128, 0,
                      acc_ref.at[pl.ds(a_keep2, 128), pl.ds(0, DHALF)],
                      px, 2)
        rb = exchange(b_keep2, 128, DHALF,
                      acc_ref.at[pl.ds(b_keep2, 128), pl.ds(DHALF, DHALF)],
                      py, 6)
        ra.wait()
        rb.wait()

        ra = exchange(a_keep1, SQ, 0,
                      acc_ref.at[pl.ds(a_keep1, SQ), pl.ds(0, DHALF)],
                      py, 3)
        rb = exchange(b_keep1, SQ, DHALF,
                      acc_ref.at[pl.ds(b_keep1, SQ), pl.ds(DHALF, DHALF)],
                      px, 7)
        out_ref[pl.ds(cy, 1), :, pl.ds(0, DHALF)] = (
            acc_ref[pl.ds(a_keep1, SQ), pl.ds(0, DHALF)].reshape(1, SQ, DHALF))
        out_ref[pl.ds(cx, 1), :, pl.ds(DHALF, DHALF)] = (
            acc_ref[pl.ds(b_keep1, SQ),
                    pl.ds(DHALF, DHALF)].reshape(1, SQ, DHALF))
        ra.wait()
        rb.wait()
        out_ref[pl.ds(1 - cy, 1), :, pl.ds(0, DHALF)] = (
            acc_ref[pl.ds(a_send1, SQ), pl.ds(0, DHALF)].reshape(1, SQ, DHALF))
        out_ref[pl.ds(1 - cx, 1), :, pl.ds(DHALF, DHALF)] = (
            acc_ref[pl.ds(b_send1, SQ),
                    pl.ds(DHALF, DHALF)].reshape(1, SQ, DHALF))

    return pl.pallas_call(
        body,
        out_shape=jax.ShapeDtypeStruct((B, SQ, D), jnp.float32),
        in_specs=[pl.BlockSpec(memory_space=pltpu.VMEM)] * 5,
        out_specs=pl.BlockSpec(memory_space=pltpu.VMEM),
        scratch_shapes=[
            pltpu.VMEM((R, HQ_PER * DH), jnp.bfloat16),
            pltpu.VMEM((R, D), jnp.float32),
            pltpu.VMEM((SQ, DHALF), jnp.float32),
            pltpu.VMEM((SQ, DHALF), jnp.float32),
            pltpu.VMEM((128, DHALF), jnp.float32),
            pltpu.VMEM((128, DHALF), jnp.float32),
            pltpu.SemaphoreType.DMA((8,)),
            pltpu.SemaphoreType.DMA((8,)),
        ],
        compiler_params=pltpu.CompilerParams(collective_id=0),
    )(x, Wq, Wo, Wk, Wv)


# device time: 28463 ns/iter; 1.2710x vs baseline; 1.2710x over previous
import jax
import jax.numpy as jnp
from jax import lax
from jax.experimental import pallas as pl
from jax.experimental.pallas import tpu as pltpu

N_DEV = 4
B, SQ, D = 2, 256, 768
HQ_PER, DH = 8, 64
KV_PER = 2
R = B * SQ
DHALF = D // 2

_MESH = pl.DeviceIdType.MESH


def kernel(x, Wq, Wo, Wk, Wv):
    def body(x_ref, wq_ref, wo_ref, wk_ref, wv_ref, out_ref,
             attn_ref, acc_ref, land_a1, land_b1, land_a2, land_b2,
             send_sems, recv_sems):
        p = lax.axis_index("i")
        cx = lax.shift_right_logical(p, 1)
        cy = lax.bitwise_and(lax.bitwise_xor(p, cx), 1)
        py = lax.bitwise_xor(p, 1)
        px = lax.bitwise_xor(p, 3)

        barrier_sem = pltpu.get_barrier_semaphore()
        for nbr in (py, px):
            pl.semaphore_signal(barrier_sem, inc=1, device_id=(nbr,),
                                device_id_type=_MESH)
        pl.semaphore_wait(barrier_sem, 2)

        x2 = x_ref[:].reshape(R, D)
        wqs = wq_ref[:] * 0.125
        q = jnp.dot(x2, wqs, preferred_element_type=jnp.float32)
        wk_sl = wk_ref[:, pl.ds(p * (KV_PER * DH), KV_PER * DH)]
        wv_sl = wv_ref[:, pl.ds(p * (KV_PER * DH), KV_PER * DH)]
        k = jnp.dot(x2, wk_sl, preferred_element_type=jnp.float32)
        v = jnp.dot(x2, wv_sl, preferred_element_type=jnp.float32)

        for b in range(B):
            rows = slice(b * SQ, (b + 1) * SQ)
            for h in range(HQ_PER):
                g = h // 4
                qh = q[rows, h * DH:(h + 1) * DH]
                kh = k[rows, g * DH:(g + 1) * DH]
                vh = v[rows, g * DH:(g + 1) * DH]
                s = lax.dot_general(
                    qh, kh, (((1,), (1,)), ((), ())),
                    preferred_element_type=jnp.float32)
                e = jnp.exp(s)
                denom = jnp.sum(e, axis=1, keepdims=True)
                o = jnp.dot(e, vh, preferred_element_type=jnp.float32) / denom
                attn_ref[rows, h * DH:(h + 1) * DH] = o

        def wo_quadrant(row_off, col_off):
            block = attn_ref[pl.ds(row_off, SQ), :]
            prod = jnp.dot(block, wo_ref[:, col_off:col_off + DHALF],
                           preferred_element_type=jnp.float32)
            return prod.astype(jnp.bfloat16)

        def exchange(src_off, n_rows, col_off, dst_ref, partner, sem):
            rdma = pltpu.make_async_remote_copy(
                src_ref=acc_ref.at[pl.ds(src_off, n_rows),
                                   pl.ds(col_off, DHALF)],
                dst_ref=dst_ref,
                send_sem=send_sems.at[sem],
                recv_sem=recv_sems.at[sem],
                device_id=(partner,),
                device_id_type=_MESH,
            )
            rdma.start()
            return rdma

        def acc_add(row_off, n_rows, col_off, land):
            acc_ref[pl.ds(row_off, n_rows), pl.ds(col_off, DHALF)] = (
                acc_ref[pl.ds(row_off, n_rows), pl.ds(col_off, DHALF)]
                + land[:, :]
            )

        a_keep1, a_send1 = cy * SQ, (1 - cy) * SQ
        b_keep1, b_send1 = cx * SQ, (1 - cx) * SQ
        a_keep2, a_send2 = a_keep1 + cx * 128, a_keep1 + (1 - cx) * 128
        b_keep2, b_send2 = b_keep1 + cy * 128, b_keep1 + (1 - cy) * 128

        acc_ref[pl.ds(a_send1, SQ), pl.ds(0, DHALF)] = wo_quadrant(a_send1, 0)
        ra = exchange(a_send1, SQ, 0, land_a1, py, 0)
        acc_ref[pl.ds(b_send1, SQ), pl.ds(DHALF, DHALF)] = (
            wo_quadrant(b_send1, DHALF))
        rb = exchange(b_send1, SQ, DHALF, land_b1, px, 4)
        keep_a = wo_quadrant(a_keep1, 0)
        keep_b = wo_quadrant(b_keep1, DHALF)

        ra.wait()
        acc_ref[pl.ds(a_keep1, SQ), pl.ds(0, DHALF)] = keep_a + land_a1[:, :]
        ra = exchange(a_send2, 128, 0, land_a2, px, 1)
        rb.wait()
        acc_ref[pl.ds(b_keep1, SQ), pl.ds(DHALF, DHALF)] = (
            keep_b + land_b1[:, :])
        rb = exchange(b_send2, 128, DHALF, land_b2, py, 5)

        ra.wait()
        acc_add(a_keep2, 128, 0, land_a2)
        ra = exchange(a_keep2, 128, 0,
                      acc_ref.at[pl.ds(a_keep2, 128), pl.ds(0, DHALF)],
                      px, 2)
        rb.wait()
        acc_add(b_keep2, 128, DHALF, land_b2)
        rb = exchange(b_keep2, 128, DHALF,
                      acc_ref.at[pl.ds(b_keep2, 128), pl.ds(DHALF, DHALF)],
                      py, 6)

        ra.wait()
        ra = exchange(a_keep1, SQ, 0,
                      acc_ref.at[pl.ds(a_keep1, SQ), pl.ds(0, DHALF)],
                      py, 3)
        rb.wait()
        rb = exchange(b_keep1, SQ, DHALF,
                      acc_ref.at[pl.ds(b_keep1, SQ), pl.ds(DHALF, DHALF)],
                      px, 7)

        out_ref[pl.ds(cy, 1), :, pl.ds(0, DHALF)] = (
            acc_ref[pl.ds(a_keep1, SQ), pl.ds(0, DHALF)]
            .astype(jnp.float32).reshape(1, SQ, DHALF))
        out_ref[pl.ds(cx, 1), :, pl.ds(DHALF, DHALF)] = (
            acc_ref[pl.ds(b_keep1, SQ), pl.ds(DHALF, DHALF)]
            .astype(jnp.float32).reshape(1, SQ, DHALF))
        ra.wait()
        out_ref[pl.ds(1 - cy, 1), :, pl.ds(0, DHALF)] = (
            acc_ref[pl.ds(a_send1, SQ), pl.ds(0, DHALF)]
            .astype(jnp.float32).reshape(1, SQ, DHALF))
        rb.wait()
        out_ref[pl.ds(1 - cx, 1), :, pl.ds(DHALF, DHALF)] = (
            acc_ref[pl.ds(b_send1, SQ), pl.ds(DHALF, DHALF)]
            .astype(jnp.float32).reshape(1, SQ, DHALF))

    return pl.pallas_call(
        body,
        out_shape=jax.ShapeDtypeStruct((B, SQ, D), jnp.float32),
        in_specs=[pl.BlockSpec(memory_space=pltpu.VMEM)] * 5,
        out_specs=pl.BlockSpec(memory_space=pltpu.VMEM),
        scratch_shapes=[
            pltpu.VMEM((R, HQ_PER * DH), jnp.float32),
            pltpu.VMEM((R, D), jnp.bfloat16),
            pltpu.VMEM((SQ, DHALF), jnp.bfloat16),
            pltpu.VMEM((SQ, DHALF), jnp.bfloat16),
            pltpu.VMEM((128, DHALF), jnp.bfloat16),
            pltpu.VMEM((128, DHALF), jnp.bfloat16),
            pltpu.SemaphoreType.DMA((8,)),
            pltpu.SemaphoreType.DMA((8,)),
        ],
        compiler_params=pltpu.CompilerParams(collective_id=0),
    )(x, Wq, Wo, Wk, Wv)
